# baseline (speedup 1.0000x reference)
import jax
import jax.numpy as jnp
from jax.experimental import pallas as pl
from jax.experimental.pallas import tpu as pltpu

BATCH = 16
SEQ = 512
IN_SIZE = 256
PROJ = 256
HID = 128
DENSE = 256
NUM_STEPS = 10
BETA = 0.95
THRESH = 1.0

FEAT = SEQ * HID
N_CHUNKS = 8
CHUNK = FEAT // N_CHUNKS


def _cur_kernel(x_ref, wp_ref, bp_ref, wg_ref, bg_ref, cur_ref):
    xb = x_ref[0]
    xp = jax.lax.dot_general(
        xb, wp_ref[...],
        dimension_numbers=(((1,), (1,)), ((), ())),
        preferred_element_type=jnp.float32) + bp_ref[...]
    xw = jax.lax.dot_general(
        xp.astype(jnp.bfloat16), wg_ref[...],
        dimension_numbers=(((1,), (1,)), ((), ())),
        preferred_element_type=jnp.float32)

    t = jax.lax.broadcasted_iota(jnp.int32, (SEQ, 1), 0)
    end = (t == 0) | (t == SEQ - 1)
    dinv = jax.lax.rsqrt(jnp.where(end, 2.0, 3.0))
    zc = jnp.zeros((1, 1), jnp.float32)
    dinv_prev = jnp.concatenate([zc, dinv[:-1]], axis=0)
    dinv_next = jnp.concatenate([dinv[1:], zc], axis=0)
    cl = dinv_prev * dinv
    cr = dinv_next * dinv
    cs = dinv * dinv

    zrow = jnp.zeros((1, HID), jnp.float32)
    xw_prev = jnp.concatenate([zrow, xw[:-1]], axis=0)
    xw_next = jnp.concatenate([xw[1:], zrow], axis=0)
    cur_ref[0] = ((cl * xw_prev + cr * xw_next) + cs * xw) + bg_ref[...]


def _dense_kernel(cur_ref, wd_ref, bd_ref, wo_ref, bo_ref, out_ref, acc_ref):
    k = pl.program_id(0)

    @pl.when(k == 0)
    def _():
        acc_ref[...] = jnp.zeros_like(acc_ref)

    cur = cur_ref[...]
    mem = cur
    for _ in range(NUM_STEPS - 1):
        reset = jnp.where(mem > THRESH, jnp.float32(THRESH), jnp.float32(0.0))
        mem = BETA * mem + cur - reset

    acc_ref[...] += jax.lax.dot_general(
        mem.astype(jnp.bfloat16), wd_ref[...],
        dimension_numbers=(((1,), (1,)), ((), ())),
        preferred_element_type=jnp.float32)

    @pl.when(k == N_CHUNKS - 1)
    def _():
        y = jnp.maximum(acc_ref[...] + bd_ref[...], 0.0)
        o = jnp.sum(y * wo_ref[...], axis=1, keepdims=True)
        out_ref[...] = jax.nn.sigmoid(o + bo_ref[0, 0])


def kernel(x, W_proj, b_proj, W_gcn, b_gcn, W_dense, b_dense, W_out, b_out):
    xb = x.astype(jnp.bfloat16)
    wpb = W_proj.astype(jnp.bfloat16)
    wgb = W_gcn.astype(jnp.bfloat16)
    wdb = W_dense.astype(jnp.bfloat16)
    bp2 = b_proj.reshape(1, PROJ)
    bg2 = b_gcn.reshape(1, HID)
    bd2 = b_dense.reshape(1, DENSE)
    bo2 = b_out.reshape(1, 1)

    cur = pl.pallas_call(
        _cur_kernel,
        grid=(BATCH,),
        in_specs=[
            pl.BlockSpec((1, SEQ, IN_SIZE), lambda b: (b, 0, 0)),
            pl.BlockSpec((PROJ, IN_SIZE), lambda b: (0, 0)),
            pl.BlockSpec((1, PROJ), lambda b: (0, 0)),
            pl.BlockSpec((HID, PROJ), lambda b: (0, 0)),
            pl.BlockSpec((1, HID), lambda b: (0, 0)),
        ],
        out_specs=pl.BlockSpec((1, SEQ, HID), lambda b: (b, 0, 0)),
        out_shape=jax.ShapeDtypeStruct((BATCH, SEQ, HID), jnp.float32),
    )(xb, wpb, bp2, wgb, bg2)

    cur2 = cur.reshape(BATCH, FEAT)

    out = pl.pallas_call(
        _dense_kernel,
        grid=(N_CHUNKS,),
        in_specs=[
            pl.BlockSpec((BATCH, CHUNK), lambda k: (0, k)),
            pl.BlockSpec((DENSE, CHUNK), lambda k: (0, k)),
            pl.BlockSpec((1, DENSE), lambda k: (0, 0)),
            pl.BlockSpec((1, DENSE), lambda k: (0, 0)),
            pl.BlockSpec(memory_space=pltpu.SMEM),
        ],
        out_specs=pl.BlockSpec((BATCH, 1), lambda k: (0, 0)),
        out_shape=jax.ShapeDtypeStruct((BATCH, 1), jnp.float32),
        scratch_shapes=[pltpu.VMEM((BATCH, DENSE), jnp.float32)],
    )(cur2, wdb, bd2, W_out, bo2)

    return out


# speedup vs baseline: 1.6940x; 1.6940x over previous
import jax
import jax.numpy as jnp
from jax.experimental import pallas as pl
from jax.experimental.pallas import tpu as pltpu

BATCH = 16
SEQ = 512
IN_SIZE = 256
PROJ = 256
HID = 128
DENSE = 256
NUM_STEPS = 10
BETA = 0.95
THRESH = 1.0

FEAT = SEQ * HID
N_CHUNKS = 8
CHUNK = FEAT // N_CHUNKS


def _cur_kernel(x_ref, wp_ref, bp_ref, wg_ref, bg_ref, cur_ref):
    xb = x_ref[0]
    xp = jax.lax.dot_general(
        xb, wp_ref[...],
        dimension_numbers=(((1,), (1,)), ((), ())),
        preferred_element_type=jnp.float32) + bp_ref[...]
    xw = jax.lax.dot_general(
        xp, wg_ref[...],
        dimension_numbers=(((1,), (1,)), ((), ())),
        preferred_element_type=jnp.float32)

    t = jax.lax.broadcasted_iota(jnp.int32, (SEQ, 1), 0)
    end = (t == 0) | (t == SEQ - 1)
    dinv = jax.lax.rsqrt(jnp.where(end, 2.0, 3.0))
    zc = jnp.zeros((1, 1), jnp.float32)
    dinv_prev = jnp.concatenate([zc, dinv[:-1]], axis=0)
    dinv_next = jnp.concatenate([dinv[1:], zc], axis=0)
    cl = dinv_prev * dinv
    cr = dinv_next * dinv
    cs = dinv * dinv

    zrow = jnp.zeros((1, HID), jnp.float32)
    xw_prev = jnp.concatenate([zrow, xw[:-1]], axis=0)
    xw_next = jnp.concatenate([xw[1:], zrow], axis=0)
    cur_ref[0] = ((cl * xw_prev + cr * xw_next) + cs * xw) + bg_ref[...]


def _dense_kernel(cur_ref, wd_ref, bd_ref, wo_ref, bo_ref, out_ref, acc_ref):
    k = pl.program_id(0)

    @pl.when(k == 0)
    def _():
        acc_ref[...] = jnp.zeros_like(acc_ref)

    cur = cur_ref[...]
    mem = cur
    for _ in range(NUM_STEPS - 1):
        reset = jnp.where(mem > THRESH, jnp.float32(THRESH), jnp.float32(0.0))
        mem = BETA * mem + cur - reset

    acc_ref[...] += jax.lax.dot_general(
        mem, wd_ref[...],
        dimension_numbers=(((1,), (1,)), ((), ())),
        preferred_element_type=jnp.float32)

    @pl.when(k == N_CHUNKS - 1)
    def _():
        y = jnp.maximum(acc_ref[...] + bd_ref[...], 0.0)
        o = jnp.sum(y * wo_ref[...], axis=1, keepdims=True)
        out_ref[...] = jax.nn.sigmoid(o + bo_ref[0, 0])


def kernel(x, W_proj, b_proj, W_gcn, b_gcn, W_dense, b_dense, W_out, b_out):
    bp2 = b_proj.reshape(1, PROJ)
    bg2 = b_gcn.reshape(1, HID)
    bd2 = b_dense.reshape(1, DENSE)
    bo2 = b_out.reshape(1, 1)

    cur = pl.pallas_call(
        _cur_kernel,
        grid=(BATCH,),
        in_specs=[
            pl.BlockSpec((1, SEQ, IN_SIZE), lambda b: (b, 0, 0)),
            pl.BlockSpec((PROJ, IN_SIZE), lambda b: (0, 0)),
            pl.BlockSpec((1, PROJ), lambda b: (0, 0)),
            pl.BlockSpec((HID, PROJ), lambda b: (0, 0)),
            pl.BlockSpec((1, HID), lambda b: (0, 0)),
        ],
        out_specs=pl.BlockSpec((1, SEQ, HID), lambda b: (b, 0, 0)),
        out_shape=jax.ShapeDtypeStruct((BATCH, SEQ, HID), jnp.float32),
    )(x, W_proj, bp2, W_gcn, bg2)

    cur2 = cur.reshape(BATCH, FEAT)

    out = pl.pallas_call(
        _dense_kernel,
        grid=(N_CHUNKS,),
        in_specs=[
            pl.BlockSpec((BATCH, CHUNK), lambda k: (0, k)),
            pl.BlockSpec((DENSE, CHUNK), lambda k: (0, k)),
            pl.BlockSpec((1, DENSE), lambda k: (0, 0)),
            pl.BlockSpec((1, DENSE), lambda k: (0, 0)),
            pl.BlockSpec(memory_space=pltpu.SMEM),
        ],
        out_specs=pl.BlockSpec((BATCH, 1), lambda k: (0, 0)),
        out_shape=jax.ShapeDtypeStruct((BATCH, 1), jnp.float32),
        scratch_shapes=[pltpu.VMEM((BATCH, DENSE), jnp.float32)],
    )(cur2, W_dense, bd2, W_out, bo2)

    return out


# speedup vs baseline: 2.0115x; 1.1874x over previous
import jax
import jax.numpy as jnp
from jax.experimental import pallas as pl
from jax.experimental.pallas import tpu as pltpu

BATCH = 16
SEQ = 512
IN_SIZE = 256
PROJ = 256
HID = 128
DENSE = 256
NUM_STEPS = 10
BETA = 0.95
THRESH = 1.0

FEAT = SEQ * HID
N_CHUNKS = 8
CHUNK = FEAT // N_CHUNKS
BGRP = 4
GROWS = BGRP * SEQ


def _cur_kernel(x_ref, wp_ref, bp_ref, wg_ref, bg_ref, cur_ref):
    xb = x_ref[...].reshape(GROWS, IN_SIZE)
    xp = jax.lax.dot_general(
        xb, wp_ref[...],
        dimension_numbers=(((1,), (1,)), ((), ())),
        preferred_element_type=jnp.float32) + bp_ref[...]
    xw = jax.lax.dot_general(
        xp, wg_ref[...],
        dimension_numbers=(((1,), (1,)), ((), ())),
        preferred_element_type=jnp.float32)

    r = jax.lax.broadcasted_iota(jnp.int32, (GROWS, 1), 0)
    t = jax.lax.rem(r, SEQ)
    first = t == 0
    last = t == SEQ - 1
    dinv = jax.lax.rsqrt(jnp.where(first | last, 2.0, 3.0))
    dinv_m1 = jax.lax.rsqrt(jnp.where((t == 1) | (t == 0), 2.0, 3.0))
    dinv_p1 = jax.lax.rsqrt(jnp.where((t == SEQ - 2) | last, 2.0, 3.0))
    cl = jnp.where(first, 0.0, dinv_m1 * dinv)
    cr = jnp.where(last, 0.0, dinv_p1 * dinv)
    cs = dinv * dinv

    zrow = jnp.zeros((1, HID), jnp.float32)
    xw_prev = jnp.concatenate([zrow, xw[:-1]], axis=0)
    xw_next = jnp.concatenate([xw[1:], zrow], axis=0)
    cur = ((cl * xw_prev + cr * xw_next) + cs * xw) + bg_ref[...]
    cur_ref[...] = cur.reshape(BGRP, SEQ, HID)


def _lif(cur):
    mem = cur
    for _ in range(NUM_STEPS - 1):
        reset = jnp.where(mem > THRESH, jnp.float32(THRESH), jnp.float32(0.0))
        mem = BETA * mem + cur - reset
    return mem


def _dense_kernel(cur_ref, wd_ref, bd_ref, wo_ref, bo_ref, out_ref, acc_ref):
    k = pl.program_id(0)

    @pl.when(k == 0)
    def _():
        acc_ref[...] = jnp.zeros_like(acc_ref)

    h = CHUNK // 2
    mem_a = _lif(cur_ref[:, :h])
    mem_b = _lif(cur_ref[:, h:])
    pa = jax.lax.dot_general(
        mem_a, wd_ref[:, :h],
        dimension_numbers=(((1,), (1,)), ((), ())),
        preferred_element_type=jnp.float32)
    pb = jax.lax.dot_general(
        mem_b, wd_ref[:, h:],
        dimension_numbers=(((1,), (1,)), ((), ())),
        preferred_element_type=jnp.float32)
    acc_ref[...] += pa + pb

    @pl.when(k == N_CHUNKS - 1)
    def _():
        y = jnp.maximum(acc_ref[...] + bd_ref[...], 0.0)
        o = jnp.sum(y * wo_ref[...], axis=1, keepdims=True)
        out_ref[...] = jax.nn.sigmoid(o + bo_ref[0, 0])


def kernel(x, W_proj, b_proj, W_gcn, b_gcn, W_dense, b_dense, W_out, b_out):
    bp2 = b_proj.reshape(1, PROJ)
    bg2 = b_gcn.reshape(1, HID)
    bd2 = b_dense.reshape(1, DENSE)
    bo2 = b_out.reshape(1, 1)

    cur = pl.pallas_call(
        _cur_kernel,
        grid=(BATCH // BGRP,),
        in_specs=[
            pl.BlockSpec((BGRP, SEQ, IN_SIZE), lambda b: (b, 0, 0)),
            pl.BlockSpec((PROJ, IN_SIZE), lambda b: (0, 0)),
            pl.BlockSpec((1, PROJ), lambda b: (0, 0)),
            pl.BlockSpec((HID, PROJ), lambda b: (0, 0)),
            pl.BlockSpec((1, HID), lambda b: (0, 0)),
        ],
        out_specs=pl.BlockSpec((BGRP, SEQ, HID), lambda b: (b, 0, 0)),
        out_shape=jax.ShapeDtypeStruct((BATCH, SEQ, HID), jnp.float32),
    )(x, W_proj, bp2, W_gcn, bg2)

    cur2 = cur.reshape(BATCH, FEAT)

    out = pl.pallas_call(
        _dense_kernel,
        grid=(N_CHUNKS,),
        in_specs=[
            pl.BlockSpec((BATCH, CHUNK), lambda k: (0, k)),
            pl.BlockSpec((DENSE, CHUNK), lambda k: (0, k)),
            pl.BlockSpec((1, DENSE), lambda k: (0, 0)),
            pl.BlockSpec((1, DENSE), lambda k: (0, 0)),
            pl.BlockSpec(memory_space=pltpu.SMEM),
        ],
        out_specs=pl.BlockSpec((BATCH, 1), lambda k: (0, 0)),
        out_shape=jax.ShapeDtypeStruct((BATCH, 1), jnp.float32),
        scratch_shapes=[pltpu.VMEM((BATCH, DENSE), jnp.float32)],
    )(cur2, W_dense, bd2, W_out, bo2)

    return out


# speedup vs baseline: 2.4816x; 1.2337x over previous
import jax
import jax.numpy as jnp
from jax.experimental import pallas as pl
from jax.experimental.pallas import tpu as pltpu

BATCH = 16
SEQ = 512
IN_SIZE = 256
PROJ = 256
HID = 128
DENSE = 256
NUM_STEPS = 10
BETA = 0.95
THRESH = 1.0

N_CHUNKS = 8
SBLK = SEQ // N_CHUNKS


def _two_matmul(v, wp_ref, bp_ref, wg_ref):
    xp = jax.lax.dot_general(
        v, wp_ref[...],
        dimension_numbers=(((1,), (1,)), ((), ())),
        preferred_element_type=jnp.float32) + bp_ref[...]
    return jax.lax.dot_general(
        xp, wg_ref[...],
        dimension_numbers=(((1,), (1,)), ((), ())),
        preferred_element_type=jnp.float32)


def _fused_kernel(xs_ref, xlo_ref, xhi_ref, wp_ref, bp_ref, wg_ref, bg_ref,
                  wd_ref, bd_ref, wo_ref, bo_ref, out_ref, acc_ref):
    k = pl.program_id(0)

    @pl.when(k == 0)
    def _():
        acc_ref[...] = jnp.zeros_like(acc_ref)

    base = k * SBLK
    xs2 = xs_ref[...].reshape(BATCH * SBLK, IN_SIZE)
    xw = _two_matmul(xs2, wp_ref, bp_ref, wg_ref)
    xw3 = xw.reshape(BATCH, SBLK, HID)
    xlw = _two_matmul(xlo_ref[:, 7, :], wp_ref, bp_ref, wg_ref)
    xrw = _two_matmul(xhi_ref[:, 0, :], wp_ref, bp_ref, wg_ref)

    xw_prev = jnp.concatenate([xlw[:, None, :], xw3[:, :-1, :]], axis=1)
    xw_next = jnp.concatenate([xw3[:, 1:, :], xrw[:, None, :]], axis=1)

    t = base + jax.lax.broadcasted_iota(jnp.int32, (1, SBLK, 1), 1)
    first = t == 0
    last = t == SEQ - 1
    dinv = jax.lax.rsqrt(jnp.where(first | last, 2.0, 3.0))
    dinv_m1 = jax.lax.rsqrt(jnp.where((t == 1) | first, 2.0, 3.0))
    dinv_p1 = jax.lax.rsqrt(jnp.where((t == SEQ - 2) | last, 2.0, 3.0))
    cl = jnp.where(first, 0.0, dinv_m1 * dinv)
    cr = jnp.where(last, 0.0, dinv_p1 * dinv)
    cs = dinv * dinv

    cur = ((cl * xw_prev + cr * xw_next) + cs * xw3) + bg_ref[...].reshape(1, 1, HID)

    mem = cur
    for _ in range(NUM_STEPS - 1):
        reset = jnp.where(mem > THRESH, jnp.float32(THRESH), jnp.float32(0.0))
        mem = BETA * mem + cur - reset

    acc_ref[...] += jax.lax.dot_general(
        mem.reshape(BATCH, SBLK * HID), wd_ref[...],
        dimension_numbers=(((1,), (1,)), ((), ())),
        preferred_element_type=jnp.float32)

    @pl.when(k == N_CHUNKS - 1)
    def _():
        y = jnp.maximum(acc_ref[...] + bd_ref[...], 0.0)
        o = jnp.sum(y * wo_ref[...], axis=1, keepdims=True)
        out_ref[...] = jax.nn.sigmoid(o + bo_ref[0, 0])


def kernel(x, W_proj, b_proj, W_gcn, b_gcn, W_dense, b_dense, W_out, b_out):
    bp2 = b_proj.reshape(1, PROJ)
    bg2 = b_gcn.reshape(1, HID)
    bd2 = b_dense.reshape(1, DENSE)
    bo2 = b_out.reshape(1, 1)


    out = pl.pallas_call(
        _fused_kernel,
        grid=(N_CHUNKS,),
        in_specs=[
            pl.BlockSpec((BATCH, SBLK, IN_SIZE), lambda k: (0, k, 0)),
            pl.BlockSpec((BATCH, 8, IN_SIZE),
                         lambda k: (0, jnp.maximum(k * (SBLK // 8) - 1, 0), 0)),
            pl.BlockSpec((BATCH, 8, IN_SIZE),
                         lambda k: (0, jnp.minimum(k * (SBLK // 8) + SBLK // 8,
                                                   SEQ // 8 - 1), 0)),
            pl.BlockSpec((PROJ, IN_SIZE), lambda k: (0, 0)),
            pl.BlockSpec((1, PROJ), lambda k: (0, 0)),
            pl.BlockSpec((HID, PROJ), lambda k: (0, 0)),
            pl.BlockSpec((1, HID), lambda k: (0, 0)),
            pl.BlockSpec((DENSE, SBLK * HID), lambda k: (0, k)),
            pl.BlockSpec((1, DENSE), lambda k: (0, 0)),
            pl.BlockSpec((1, DENSE), lambda k: (0, 0)),
            pl.BlockSpec(memory_space=pltpu.SMEM),
        ],
        out_specs=pl.BlockSpec((BATCH, 1), lambda k: (0, 0)),
        out_shape=jax.ShapeDtypeStruct((BATCH, 1), jnp.float32),
        scratch_shapes=[pltpu.VMEM((BATCH, DENSE), jnp.float32)],
    )(x, x, x, W_proj, bp2, W_gcn, bg2, W_dense, bd2, W_out, bo2)

    return out


# speedup vs baseline: 2.5286x; 1.0189x over previous
import jax
import jax.numpy as jnp
from jax.experimental import pallas as pl
from jax.experimental.pallas import tpu as pltpu

BATCH = 16
SEQ = 512
IN_SIZE = 256
PROJ = 256
HID = 128
DENSE = 256
NUM_STEPS = 10
BETA = 0.95
THRESH = 1.0

N_CHUNKS = 4
SBLK = SEQ // N_CHUNKS


def _two_matmul(v, wp_ref, bp_ref, wg_ref):
    xp = jax.lax.dot_general(
        v, wp_ref[...],
        dimension_numbers=(((1,), (1,)), ((), ())),
        preferred_element_type=jnp.float32) + bp_ref[...]
    return jax.lax.dot_general(
        xp, wg_ref[...],
        dimension_numbers=(((1,), (1,)), ((), ())),
        preferred_element_type=jnp.float32)


def _fused_kernel(xs_ref, xlo_ref, xhi_ref, wp_ref, bp_ref, wg_ref, bg_ref,
                  wd_ref, bd_ref, wo_ref, bo_ref, out_ref, acc_ref):
    k = pl.program_id(0)

    @pl.when(k == 0)
    def _():
        acc_ref[...] = jnp.zeros_like(acc_ref)

    base = k * SBLK
    xs2 = xs_ref[...].reshape(BATCH * SBLK, IN_SIZE)
    xw = _two_matmul(xs2, wp_ref, bp_ref, wg_ref)
    xw3 = xw.reshape(BATCH, SBLK, HID)
    xlw = _two_matmul(xlo_ref[:, 7, :], wp_ref, bp_ref, wg_ref)
    xrw = _two_matmul(xhi_ref[:, 0, :], wp_ref, bp_ref, wg_ref)

    xw_prev = jnp.concatenate([xlw[:, None, :], xw3[:, :-1, :]], axis=1)
    xw_next = jnp.concatenate([xw3[:, 1:, :], xrw[:, None, :]], axis=1)

    t = base + jax.lax.broadcasted_iota(jnp.int32, (1, SBLK, 1), 1)
    first = t == 0
    last = t == SEQ - 1
    dinv = jax.lax.rsqrt(jnp.where(first | last, 2.0, 3.0))
    dinv_m1 = jax.lax.rsqrt(jnp.where((t == 1) | first, 2.0, 3.0))
    dinv_p1 = jax.lax.rsqrt(jnp.where((t == SEQ - 2) | last, 2.0, 3.0))
    cl = jnp.where(first, 0.0, dinv_m1 * dinv)
    cr = jnp.where(last, 0.0, dinv_p1 * dinv)
    cs = dinv * dinv

    cur = ((cl * xw_prev + cr * xw_next) + cs * xw3) + bg_ref[...].reshape(1, 1, HID)

    mem = cur
    for _ in range(NUM_STEPS - 1):
        reset = jnp.where(mem > THRESH, jnp.float32(THRESH), jnp.float32(0.0))
        mem = BETA * mem + cur - reset

    acc_ref[...] += jax.lax.dot_general(
        mem.reshape(BATCH, SBLK * HID), wd_ref[...],
        dimension_numbers=(((1,), (1,)), ((), ())),
        preferred_element_type=jnp.float32)

    @pl.when(k == N_CHUNKS - 1)
    def _():
        y = jnp.maximum(acc_ref[...] + bd_ref[...], 0.0)
        o = jnp.sum(y * wo_ref[...], axis=1, keepdims=True)
        out_ref[...] = jax.nn.sigmoid(o + bo_ref[0, 0])


def kernel(x, W_proj, b_proj, W_gcn, b_gcn, W_dense, b_dense, W_out, b_out):
    bp2 = b_proj.reshape(1, PROJ)
    bg2 = b_gcn.reshape(1, HID)
    bd2 = b_dense.reshape(1, DENSE)
    bo2 = b_out.reshape(1, 1)


    out = pl.pallas_call(
        _fused_kernel,
        grid=(N_CHUNKS,),
        in_specs=[
            pl.BlockSpec((BATCH, SBLK, IN_SIZE), lambda k: (0, k, 0)),
            pl.BlockSpec((BATCH, 8, IN_SIZE),
                         lambda k: (0, jnp.maximum(k * (SBLK // 8) - 1, 0), 0)),
            pl.BlockSpec((BATCH, 8, IN_SIZE),
                         lambda k: (0, jnp.minimum(k * (SBLK // 8) + SBLK // 8,
                                                   SEQ // 8 - 1), 0)),
            pl.BlockSpec((PROJ, IN_SIZE), lambda k: (0, 0)),
            pl.BlockSpec((1, PROJ), lambda k: (0, 0)),
            pl.BlockSpec((HID, PROJ), lambda k: (0, 0)),
            pl.BlockSpec((1, HID), lambda k: (0, 0)),
            pl.BlockSpec((DENSE, SBLK * HID), lambda k: (0, k)),
            pl.BlockSpec((1, DENSE), lambda k: (0, 0)),
            pl.BlockSpec((1, DENSE), lambda k: (0, 0)),
            pl.BlockSpec(memory_space=pltpu.SMEM),
        ],
        out_specs=pl.BlockSpec((BATCH, 1), lambda k: (0, 0)),
        out_shape=jax.ShapeDtypeStruct((BATCH, 1), jnp.float32),
        scratch_shapes=[pltpu.VMEM((BATCH, DENSE), jnp.float32)],
    )(x, x, x, W_proj, bp2, W_gcn, bg2, W_dense, bd2, W_out, bo2)

    return out
